# baseline (speedup 1.0000x reference)
import functools

import jax
import jax.numpy as jnp
from jax import lax
from jax.experimental import pallas as pl
from jax.experimental.pallas import tpu as pltpu
from jax.experimental.pallas import tpu_sc as plsc

N_NODES = 10000
N_EDGES = 320000
D = 128

NC = 2
NS = 16
NW = NC * NS
LANES = 16

B = 80
NB = 128
E_PAD = NW * NB * B
WINDOW = 16
NUM_WINDOWS = NB // WINDOW
ROW_CHUNK = 80
NUM_ROW_CHUNKS = N_NODES // ROW_CHUNK


def _mm_combine_body(p_ref, w_ref, b_ref, o_ref):
    s = p_ref[0] + p_ref[1]
    o_ref[...] = jnp.dot(s, w_ref[...],
                         preferred_element_type=jnp.float32) + b_ref[...]


def _tc_mm_combine(partials, w, bias2d):
    return pl.pallas_call(
        _mm_combine_body,
        out_shape=jax.ShapeDtypeStruct((N_NODES, D), jnp.float32),
    )(partials, w, bias2d)


def _spmm_sc(x, src4, dst4, vals4):
    mesh = plsc.VectorSubcoreMesh(core_axis_name="c", subcore_axis_name="s")

    @functools.partial(
        pl.kernel,
        out_type=jax.ShapeDtypeStruct((NC, N_NODES, D), jnp.float32),
        mesh=mesh,
        scratch_types=[
            pltpu.VMEM((WINDOW, B), jnp.int32),
            pltpu.VMEM((WINDOW, B), jnp.int32),
            pltpu.VMEM((WINDOW, B), jnp.float32),
            pltpu.VMEM((B, D), jnp.float32),
            pltpu.VMEM((B, D), jnp.float32),
            pltpu.VMEM((B, D), jnp.float32),
            pltpu.VMEM((B, D), jnp.float32),
            pltpu.VMEM_SHARED((N_NODES, D), jnp.float32),
            pltpu.SemaphoreType.DMA((4,)),
            pltpu.SemaphoreType.DMA((4,)),
        ],
    )
    def k(x_hbm, src_hbm, dst_hbm, vals_hbm, out_hbm,
          src_v, dst_v, vals_v, rows0, rows1, rows2, rows3, acc_sh,
          sem_g, sem_s):
        cid = lax.axis_index("c")
        sid = lax.axis_index("s")
        wid = cid * NS + sid

        rows = (rows0, rows1, rows2, rows3)

        zvec = jnp.zeros((LANES,), jnp.float32)

        @pl.loop(0, B)
        def _(i):
            for c in range(D // LANES):
                rows0[i, pl.ds(c * LANES, LANES)] = zvec

        @pl.loop(sid, NUM_ROW_CHUNKS, step=NS)
        def _(r):
            pltpu.sync_copy(rows0, acc_sh.at[pl.ds(r * ROW_CHUNK,
                                                   ROW_CHUNK)])

        plsc.subcore_barrier()

        def start_gather(j, b):
            pltpu.async_copy(x_hbm.at[src_v.at[j]], rows[b], sem_g.at[b])

        def wait_gather(b):
            pltpu.make_async_copy(x_hbm.at[pl.ds(0, B)], rows[b],
                                  sem_g.at[b]).wait()

        def start_scatter(j, b):
            pltpu.async_copy(rows[b], acc_sh.at[dst_v.at[j]], sem_s.at[b],
                             add=True)

        def wait_scatter(b):
            pltpu.make_async_copy(rows[b], acc_sh.at[pl.ds(0, B)],
                                  sem_s.at[b]).wait()

        def scale(b, j):
            buf = rows[b]

            @pl.loop(0, B, step=LANES)
            def _(g):
                vvec = vals_v[j, pl.ds(g, LANES)]
                for i in range(LANES):
                    v = vvec[i]
                    for c in range(D // LANES):
                        sl = pl.ds(c * LANES, LANES)
                        buf[g + i, sl] = buf[g + i, sl] * v

        @pl.loop(0, NUM_WINDOWS)
        def _(w):
            pltpu.sync_copy(src_hbm.at[wid, w], src_v)
            pltpu.sync_copy(dst_hbm.at[wid, w], dst_v)
            pltpu.sync_copy(vals_hbm.at[wid, w], vals_v)

            start_gather(0, 0)
            start_gather(1, 1)

            @pl.loop(0, WINDOW // 4)
            def _(g4):
                jb = 4 * g4
                for t in range(4):
                    j = jb + t
                    b = t
                    bn = (t + 2) % 4
                    wait_gather(b)
                    scale(b, j)
                    start_scatter(j, b)

                    @pl.when(jnp.logical_and(j >= 2, j + 2 < WINDOW))
                    def _():
                        wait_scatter(bn)

                    @pl.when(j + 2 < WINDOW)
                    def _():
                        start_gather(j + 2, bn)

            wait_scatter(0)
            wait_scatter(1)
            wait_scatter(2)
            wait_scatter(3)

        plsc.subcore_barrier()

        @pl.loop(sid, NUM_ROW_CHUNKS, step=NS)
        def _(r):
            dbase = r * ROW_CHUNK
            pltpu.sync_copy(acc_sh.at[pl.ds(dbase, ROW_CHUNK)],
                            out_hbm.at[cid, pl.ds(dbase, ROW_CHUNK)])

    return k(x, src4, dst4, vals4)


def kernel(sparse_poly_edge_index, sparse_poly_values, input_feature,
           weight, bias):
    dst = sparse_poly_edge_index[0].astype(jnp.int32)
    src = sparse_poly_edge_index[1].astype(jnp.int32)
    pad = E_PAD - N_EDGES
    zi = jnp.zeros((pad,), jnp.int32)
    src4 = jnp.concatenate([src, zi]).reshape(NW, NUM_WINDOWS, WINDOW, B)
    dst4 = jnp.concatenate([dst, zi]).reshape(NW, NUM_WINDOWS, WINDOW, B)
    vals4 = jnp.concatenate(
        [sparse_poly_values, jnp.zeros((pad,), jnp.float32)]
    ).reshape(NW, NUM_WINDOWS, WINDOW, B)
    partials = _spmm_sc(input_feature, src4, dst4, vals4)
    return _tc_mm_combine(partials, weight, bias.reshape(1, D))


# speedup vs baseline: 3.1465x; 3.1465x over previous
import functools

import jax
import jax.numpy as jnp
from jax import lax
from jax.experimental import pallas as pl
from jax.experimental.pallas import tpu as pltpu
from jax.experimental.pallas import tpu_sc as plsc

N_NODES = 10000
N_EDGES = 320000
D = 128

NC = 2
NS = 16
NW = NC * NS
LANES = 16

B = 80
BATCHES_PER_WORKER = N_EDGES // (NW * B)
WINDOW = 25
NUM_WINDOWS = BATCHES_PER_WORKER // WINDOW
ROW_CHUNK = 80
NUM_ROW_CHUNKS = N_NODES // ROW_CHUNK


def _mm_combine_body(p_ref, w_ref, b_ref, o_ref):
    s = p_ref[0] + p_ref[1]
    o_ref[...] = jnp.dot(s, w_ref[...],
                         preferred_element_type=jnp.float32) + b_ref[...]


def _tc_mm_combine(partials, w, bias2d):
    return pl.pallas_call(
        _mm_combine_body,
        out_shape=jax.ShapeDtypeStruct((N_NODES, D), jnp.float32),
    )(partials, w, bias2d)


def _spmm_sc(support, src3, dst3, vals3):
    mesh = plsc.VectorSubcoreMesh(core_axis_name="c", subcore_axis_name="s")

    @functools.partial(
        pl.kernel,
        out_type=jax.ShapeDtypeStruct((NC, N_NODES, D), jnp.float32),
        mesh=mesh,
        scratch_types=[
            pltpu.VMEM((WINDOW, B), jnp.int32),
            pltpu.VMEM((WINDOW, B), jnp.int32),
            pltpu.VMEM((WINDOW, B), jnp.float32),
            pltpu.VMEM((B, D), jnp.float32),
            pltpu.VMEM((B, D), jnp.float32),
            pltpu.VMEM((B, D), jnp.float32),
            pltpu.VMEM_SHARED((N_NODES, D), jnp.float32),
            pltpu.SemaphoreType.DMA,
            pltpu.SemaphoreType.DMA,
            pltpu.SemaphoreType.DMA,
            pltpu.SemaphoreType.DMA,
            pltpu.SemaphoreType.DMA,
            pltpu.SemaphoreType.DMA,
        ],
    )
    def k(support_hbm, src_hbm, dst_hbm, vals_hbm, out_hbm,
          src_v, dst_v, vals_v, rows0, rows1, rows2, acc_sh,
          sg0, sg1, sg2, ss0, ss1, ss2):
        cid = lax.axis_index("c")
        sid = lax.axis_index("s")
        wid = cid * NS + sid

        zvec = jnp.zeros((LANES,), jnp.float32)

        @pl.loop(0, B)
        def _(i):
            for c in range(D // LANES):
                rows0[i, pl.ds(c * LANES, LANES)] = zvec

        @pl.loop(sid, NUM_ROW_CHUNKS, step=NS)
        def _(r):
            pltpu.sync_copy(rows0, acc_sh.at[pl.ds(r * ROW_CHUNK, ROW_CHUNK)])

        plsc.subcore_barrier()

        rows = (rows0, rows1, rows2)
        sg = (sg0, sg1, sg2)
        ss = (ss0, ss1, ss2)

        def scale(buf, j):
            @pl.loop(0, B, step=LANES)
            def _(g):
                vvec = vals_v[j, pl.ds(g, LANES)]
                for i in range(LANES):
                    v = vvec[i]
                    for c in range(D // LANES):
                        sl = pl.ds(c * LANES, LANES)
                        buf[g + i, sl] = buf[g + i, sl] * v

        def start_gather(j, b):
            pltpu.async_copy(support_hbm.at[src_v.at[j]], rows[b], sg[b])

        def wait_gather(b):
            pltpu.make_async_copy(support_hbm.at[pl.ds(0, B)], rows[b],
                                  sg[b]).wait()

        def start_scatter(j, b):
            pltpu.async_copy(rows[b], acc_sh.at[dst_v.at[j]], ss[b],
                             add=True)

        def wait_scatter(b):
            pltpu.make_async_copy(rows[b], acc_sh.at[pl.ds(0, B)],
                                  ss[b]).wait()

        @pl.loop(0, NUM_WINDOWS)
        def _(w):
            pltpu.sync_copy(src_hbm.at[wid, w], src_v)
            pltpu.sync_copy(dst_hbm.at[wid, w], dst_v)
            pltpu.sync_copy(vals_hbm.at[wid, w], vals_v)

            start_gather(0, 0)
            start_gather(1, 1)
            start_gather(2, 2)

            wait_gather(0)
            scale(rows0, 0)
            start_scatter(0, 0)

            @pl.loop(0, (WINDOW - 1) // 3)
            def _(i):
                jb = 1 + 3 * i
                for t in range(3):
                    b = (1 + t) % 3
                    wait_gather(b)
                    scale(rows[b], jb + t)
                    start_scatter(jb + t, b)
                    nxt = jb + t + 2
                    bn = t

                    @pl.when(nxt < WINDOW)
                    def _():
                        wait_scatter(bn)
                        start_gather(nxt, bn)

            wait_scatter((WINDOW - 3) % 3)
            wait_scatter((WINDOW - 2) % 3)
            wait_scatter((WINDOW - 1) % 3)

        plsc.subcore_barrier()

        @pl.loop(sid, NUM_ROW_CHUNKS, step=NS)
        def _(r):
            dbase = r * ROW_CHUNK
            pltpu.sync_copy(acc_sh.at[pl.ds(dbase, ROW_CHUNK)],
                            out_hbm.at[cid, pl.ds(dbase, ROW_CHUNK)])

    return k(support, src3, dst3, vals3)


def kernel(sparse_poly_edge_index, sparse_poly_values, input_feature,
           weight, bias):
    dst = sparse_poly_edge_index[0].astype(jnp.int32)
    src = sparse_poly_edge_index[1].astype(jnp.int32)
    src3 = src.reshape(NW, NUM_WINDOWS, WINDOW, B)
    dst3 = dst.reshape(NW, NUM_WINDOWS, WINDOW, B)
    vals3 = sparse_poly_values.reshape(NW, NUM_WINDOWS, WINDOW, B)
    partials = _spmm_sc(input_feature, src3, dst3, vals3)
    return _tc_mm_combine(partials, weight, bias.reshape(1, D))
